# baseline (speedup 1.0000x reference)
import functools

import jax
import jax.numpy as jnp
from jax import lax
from jax.experimental import pallas as pl
from jax.experimental.pallas import tpu as pltpu
from jax.experimental.pallas import tpu_sc as plsc

NC = 2
NS = 16
NW = NC * NS
CHUNK = 128
IB = 4


def _sc_accumulate(x, row, col, zblk, oblk, n_acc, cpt, rpt):
    n, d = x.shape
    rpt_chunks = rpt // CHUNK
    mesh = plsc.VectorSubcoreMesh(core_axis_name="c", subcore_axis_name="s")

    @functools.partial(
        pl.kernel,
        out_type=[
            jax.ShapeDtypeStruct((NC, n_acc, d), jnp.float32),
            jax.ShapeDtypeStruct((NC, n_acc, d), jnp.float32),
        ],
        mesh=mesh,
        scratch_types=[
            pltpu.VMEM_SHARED((n_acc, d), jnp.float32),
            pltpu.VMEM((IB * CHUNK,), jnp.int32),
            pltpu.VMEM((IB * CHUNK,), jnp.int32),
            pltpu.VMEM((CHUNK, d), jnp.float32),
            pltpu.VMEM((CHUNK, d), jnp.float32),
            pltpu.SemaphoreType.DMA,
        ],
    )
    def k(x_hbm, row_hbm, col_hbm, z_hbm, o_hbm, sum_hbm, cnt_hbm,
          acc_sh, rb_v, cb_v, msg_v, ones_v, sem):
        cid = lax.axis_index("c")
        sid = lax.axis_index("s")
        wid = cid * NS + sid
        base_row = sid * rpt

        pltpu.sync_copy(o_hbm, ones_v)

        def zero_own_rows():
            pltpu.sync_copy(z_hbm, msg_v)
            for kk in range(rpt_chunks):
                pltpu.sync_copy(
                    msg_v, acc_sh.at[pl.ds(base_row + kk * CHUNK, CHUNK)])

        def drain_own_rows(dst_hbm):
            for kk in range(rpt_chunks):
                r0 = base_row + kk * CHUNK
                pltpu.sync_copy(acc_sh.at[pl.ds(r0, CHUNK)], msg_v)
                pltpu.sync_copy(msg_v, dst_hbm.at[cid, pl.ds(r0, CHUNK)])

        zero_own_rows()
        plsc.subcore_barrier()

        ebase = wid * cpt * CHUNK

        def body(g, _):
            e0 = ebase + g * IB * CHUNK
            pltpu.sync_copy(row_hbm.at[pl.ds(e0, IB * CHUNK)], rb_v)
            pltpu.sync_copy(col_hbm.at[pl.ds(e0, IB * CHUNK)], cb_v)
            for j in range(IB):
                ridx = rb_v.at[pl.ds(j * CHUNK, CHUNK)]
                cidx = cb_v.at[pl.ds(j * CHUNK, CHUNK)]
                pltpu.async_copy(x_hbm.at[ridx], msg_v, sem).wait()
                pltpu.sync_copy(msg_v, acc_sh.at[cidx], add=True)
            return 0
        lax.fori_loop(0, cpt // IB, body, 0)
        plsc.subcore_barrier()

        drain_own_rows(sum_hbm)
        zero_own_rows()
        plsc.subcore_barrier()

        def body2(g, _):
            e0 = ebase + g * IB * CHUNK
            pltpu.sync_copy(col_hbm.at[pl.ds(e0, IB * CHUNK)], cb_v)
            for j in range(IB):
                pltpu.sync_copy(
                    ones_v, acc_sh.at[cb_v.at[pl.ds(j * CHUNK, CHUNK)]],
                    add=True)
            return 0
        lax.fori_loop(0, cpt // IB, body2, 0)
        plsc.subcore_barrier()
        drain_own_rows(cnt_hbm)

    return k(x, row, col, zblk, oblk)


def _combine_body(sum_ref, cnt_ref, out_ref):
    s = sum_ref[0] + sum_ref[1]
    c = cnt_ref[0] + cnt_ref[1]
    out_ref[...] = s / jnp.maximum(c, 1.0)


def kernel(x, edge_index):
    n, d = x.shape
    e = edge_index.shape[1]
    row = edge_index[0].astype(jnp.int32)
    col = edge_index[1].astype(jnp.int32)

    chunks = -(-e // CHUNK)
    cpt = -(-chunks // NW)
    cpt = -(-cpt // IB) * IB
    e_pad = cpt * NW * CHUNK
    row = jnp.pad(row, (0, e_pad - e))
    col = jnp.pad(col, (0, e_pad - e), constant_values=n)

    rpt = -(-(n + 1) // (NS * CHUNK)) * CHUNK
    n_acc = NS * rpt

    zblk = jnp.zeros((CHUNK, d), jnp.float32)
    oblk = jnp.ones((CHUNK, d), jnp.float32)
    sums, cnt = _sc_accumulate(x, row, col, zblk, oblk, n_acc, cpt, rpt)

    rb = 400 if n % 400 == 0 else 8
    out = pl.pallas_call(
        _combine_body,
        out_shape=jax.ShapeDtypeStruct((n, d), jnp.float32),
        grid=(n // rb,),
        in_specs=[
            pl.BlockSpec((NC, rb, d), lambda i: (0, i, 0)),
            pl.BlockSpec((NC, rb, d), lambda i: (0, i, 0)),
        ],
        out_specs=pl.BlockSpec((rb, d), lambda i: (i, 0)),
    )(sums[:, :n], cnt[:, :n])
    return out


# speedup vs baseline: 1.2302x; 1.2302x over previous
import functools

import jax
import jax.numpy as jnp
from jax import lax
from jax.experimental import pallas as pl
from jax.experimental.pallas import tpu as pltpu
from jax.experimental.pallas import tpu_sc as plsc

NC = 2
NS = 16
NW = NC * NS
CHUNK = 128
IB = 1


def _sc_accumulate(x, row, col, zblk, oblk, n_acc, cpt, rpt):
    n, d = x.shape
    rpt_chunks = rpt // CHUNK
    mesh = plsc.VectorSubcoreMesh(core_axis_name="c", subcore_axis_name="s")

    @functools.partial(
        pl.kernel,
        out_type=[
            jax.ShapeDtypeStruct((NC, n_acc, d), jnp.float32),
            jax.ShapeDtypeStruct((NC, n_acc, d), jnp.float32),
        ],
        mesh=mesh,
        scratch_types=[
            pltpu.VMEM_SHARED((n_acc, d), jnp.float32),
            pltpu.VMEM((IB * CHUNK,), jnp.int32),
            pltpu.VMEM((IB * CHUNK,), jnp.int32),
            pltpu.VMEM((CHUNK, d), jnp.float32),
            pltpu.VMEM((CHUNK, d), jnp.float32),
            pltpu.SemaphoreType.DMA,
        ],
    )
    def k(x_hbm, row_hbm, col_hbm, z_hbm, o_hbm, sum_hbm, cnt_hbm,
          acc_sh, rb_v, cb_v, msg_v, ones_v, sem):
        cid = lax.axis_index("c")
        sid = lax.axis_index("s")
        wid = cid * NS + sid
        base_row = sid * rpt

        pltpu.sync_copy(o_hbm, ones_v)

        def zero_own_rows():
            pltpu.sync_copy(z_hbm, msg_v)
            for kk in range(rpt_chunks):
                pltpu.sync_copy(
                    msg_v, acc_sh.at[pl.ds(base_row + kk * CHUNK, CHUNK)])

        def drain_own_rows(dst_hbm):
            for kk in range(rpt_chunks):
                r0 = base_row + kk * CHUNK
                pltpu.sync_copy(acc_sh.at[pl.ds(r0, CHUNK)], msg_v)
                pltpu.sync_copy(msg_v, dst_hbm.at[cid, pl.ds(r0, CHUNK)])

        zero_own_rows()
        plsc.subcore_barrier()

        ebase = wid * cpt * CHUNK

        def body(g, _):
            e0 = ebase + g * IB * CHUNK
            pltpu.sync_copy(row_hbm.at[pl.ds(e0, IB * CHUNK)], rb_v)
            pltpu.sync_copy(col_hbm.at[pl.ds(e0, IB * CHUNK)], cb_v)
            for j in range(IB):
                ridx = rb_v.at[pl.ds(j * CHUNK, CHUNK)]
                cidx = cb_v.at[pl.ds(j * CHUNK, CHUNK)]
                pltpu.async_copy(x_hbm.at[ridx], msg_v, sem).wait()
                pltpu.sync_copy(msg_v, acc_sh.at[cidx], add=True)
            return 0
        lax.fori_loop(0, cpt // IB, body, 0)
        plsc.subcore_barrier()

        drain_own_rows(sum_hbm)
        zero_own_rows()
        plsc.subcore_barrier()

        def body2(g, _):
            e0 = ebase + g * IB * CHUNK
            pltpu.sync_copy(col_hbm.at[pl.ds(e0, IB * CHUNK)], cb_v)
            for j in range(IB):
                pltpu.sync_copy(
                    ones_v, acc_sh.at[cb_v.at[pl.ds(j * CHUNK, CHUNK)]],
                    add=True)
            return 0
        lax.fori_loop(0, cpt // IB, body2, 0)
        plsc.subcore_barrier()
        drain_own_rows(cnt_hbm)

    return k(x, row, col, zblk, oblk)


def _combine_body(sum_ref, cnt_ref, out_ref):
    s = sum_ref[0] + sum_ref[1]
    c = cnt_ref[0] + cnt_ref[1]
    out_ref[...] = s / jnp.maximum(c, 1.0)


def kernel(x, edge_index):
    n, d = x.shape
    e = edge_index.shape[1]
    row = edge_index[0].astype(jnp.int32)
    col = edge_index[1].astype(jnp.int32)

    chunks = -(-e // CHUNK)
    cpt = -(-chunks // NW)
    cpt = -(-cpt // IB) * IB
    e_pad = cpt * NW * CHUNK
    row = jnp.pad(row, (0, e_pad - e))
    col = jnp.pad(col, (0, e_pad - e), constant_values=n)

    rpt = -(-(n + 1) // (NS * CHUNK)) * CHUNK
    n_acc = NS * rpt

    zblk = jnp.zeros((CHUNK, d), jnp.float32)
    oblk = jnp.ones((CHUNK, d), jnp.float32)
    sums, cnt = _sc_accumulate(x, row, col, zblk, oblk, n_acc, cpt, rpt)

    rb = 400 if n % 400 == 0 else 8
    out = pl.pallas_call(
        _combine_body,
        out_shape=jax.ShapeDtypeStruct((n, d), jnp.float32),
        grid=(n // rb,),
        in_specs=[
            pl.BlockSpec((NC, rb, d), lambda i: (0, i, 0)),
            pl.BlockSpec((NC, rb, d), lambda i: (0, i, 0)),
        ],
        out_specs=pl.BlockSpec((rb, d), lambda i: (i, 0)),
    )(sums[:, :n], cnt[:, :n])
    return out
